# baseline (speedup 1.0000x reference)
import functools

import jax
import jax.numpy as jnp
from jax import lax
from jax.experimental import pallas as pl
from jax.experimental.pallas import tpu as pltpu
from jax.experimental.pallas import tpu_sc as plsc

_SC_INFO = plsc.get_sparse_core_info()
_NC = _SC_INFO.num_cores
_NS = _SC_INFO.num_subcores
_NW = _NC * _NS

_CHUNK_A = 16
_CHUNK_B = 8
_NBUF_A = 3
_NBUF_B = 2


def _make_sc_copy(S, D, dtype):
    rows_per_w = S // _NW
    half = rows_per_w // 2
    nsteps_a = half // _CHUNK_A
    nsteps_b = half // _CHUNK_B

    mesh = plsc.VectorSubcoreMesh(core_axis_name="c", subcore_axis_name="s")

    @functools.partial(
        pl.kernel,
        mesh=mesh,
        out_type=jax.ShapeDtypeStruct((1, S, D), dtype),
        scratch_types=[
            pltpu.VMEM((_NBUF_A, _CHUNK_A, D), dtype),
            pltpu.VMEM_SHARED((_NS, _NBUF_B, _CHUNK_B, D), dtype),
            pltpu.SemaphoreType.DMA((_NBUF_A,)),
            pltpu.SemaphoreType.DMA((_NBUF_A,)),
            pltpu.SemaphoreType.DMA((_NBUF_B,)),
            pltpu.SemaphoreType.DMA((_NBUF_B,)),
        ],
    )
    def sc_copy(w_hbm, o_hbm, tbuf, sbuf, ain_sem, aout_sem, bin_sem, bout_sem):
        cid = lax.axis_index("c")
        sid = lax.axis_index("s")
        wid = sid * _NC + cid
        base_a = wid * rows_per_w
        base_b = base_a + half

        def a_in(step, slot):
            return pltpu.make_async_copy(
                w_hbm.at[pl.ds(base_a + step * _CHUNK_A, _CHUNK_A)],
                tbuf.at[slot],
                ain_sem.at[slot],
            )

        def a_out(step, slot):
            return pltpu.make_async_copy(
                tbuf.at[slot],
                o_hbm.at[0, pl.ds(base_a + step * _CHUNK_A, _CHUNK_A)],
                aout_sem.at[slot],
            )

        def b_in(step, slot):
            return pltpu.make_async_copy(
                w_hbm.at[pl.ds(base_b + step * _CHUNK_B, _CHUNK_B)],
                sbuf.at[sid, slot],
                bin_sem.at[slot],
            )

        def b_out(step, slot):
            return pltpu.make_async_copy(
                sbuf.at[sid, slot],
                o_hbm.at[0, pl.ds(base_b + step * _CHUNK_B, _CHUNK_B)],
                bout_sem.at[slot],
            )

        for s in range(min(_NBUF_A, nsteps_a)):
            a_in(s, s).start()
        for s in range(min(_NBUF_B, nsteps_b)):
            b_in(s, s).start()

        def adv_a(step):
            slot = step % _NBUF_A
            a_in(step, slot).wait()
            a_out(step, slot).start()
            nxt = step + _NBUF_A
            if nxt < nsteps_a:
                a_out(step, slot).wait()
                a_in(nxt, slot).start()

        def adv_b(step):
            slot = step % _NBUF_B
            b_in(step, slot).wait()
            b_out(step, slot).start()
            nxt = step + _NBUF_B
            if nxt < nsteps_b:
                b_out(step, slot).wait()
                b_in(nxt, slot).start()

        for i in range(nsteps_b):
            adv_b(i)
            if i % 2 == 0:
                adv_a(i // 2)
        for step in range(max(nsteps_a - _NBUF_A, 0), nsteps_a):
            a_out(step, step % _NBUF_A).wait()
        for step in range(max(nsteps_b - _NBUF_B, 0), nsteps_b):
            b_out(step, step % _NBUF_B).wait()

    return sc_copy


def kernel(embedding_weight, seq_len):
    del seq_len
    S, D = embedding_weight.shape
    return _make_sc_copy(S, D, embedding_weight.dtype)(embedding_weight)
